# baseline (speedup 1.0000x reference)
import jax
import jax.numpy as jnp
from jax import lax
from jax.experimental import pallas as pl
from jax.experimental.pallas import tpu as pltpu
from jax.experimental.pallas import tpu_sc as plsc


def _top3_t(d):
    s, n = d.shape
    iota = lax.broadcasted_iota(jnp.int32, (s, n), 0)
    fbig = jnp.float32(3.0e38)
    sbig = jnp.int32(2 * s)
    dwork = d
    idxs, recips, sels = [], [], []
    for k in range(3):
        m = jnp.min(dwork, axis=0, keepdims=True)
        elig = dwork == m
        idxm = jnp.min(jnp.where(elig, iota, sbig), axis=0, keepdims=True)
        sel = iota == idxm
        idxs.append(idxm)
        sels.append(sel)
        recips.append(1.0 / (jnp.maximum(m, 0.0) + 1e-8))
        if k < 2:
            dwork = jnp.where(sel, fbig, dwork)
    rnorm = 1.0 / (recips[0] + recips[1] + recips[2])
    ws = [r * rnorm for r in recips]
    return idxs, ws, sels


def _weight_matrix(sels, ws, like):
    w = jnp.where(sels[0], ws[0], jnp.zeros_like(like))
    w = jnp.where(sels[1], ws[1], w)
    w = jnp.where(sels[2], ws[2], w)
    return w


def _sqdist_t(r, qt):
    rr = jnp.sum(r * r, axis=-1, keepdims=True)
    qq = jnp.sum(qt * qt, axis=0, keepdims=True)
    rq = jnp.dot(r, qt, preferred_element_type=jnp.float32)
    return rr + qq - 2.0 * rq


def _select_body(x1t_ref, xyz2_ref, xt_ref, w1t_ref, b1t_ref,
                 x0t_ref, xyz1_ref, h1t_ref, idx_ref, wts_ref):
    j = pl.program_id(1)

    @pl.when(j == 0)
    def _stage1():
        d = _sqdist_t(xyz2_ref[0], x1t_ref[0])
        _, ws, sels = _top3_t(d)
        w = _weight_matrix(sels, ws, d)
        interp = jnp.dot(xt_ref[0], w, preferred_element_type=jnp.float32)
        h = jnp.dot(w1t_ref[...], interp, preferred_element_type=jnp.float32)
        h1t_ref[0] = jnp.maximum(h + b1t_ref[...], 0.0)

    d = _sqdist_t(xyz1_ref[0], x0t_ref[0])
    idxs, ws, _ = _top3_t(d)
    idx_ref[0, 0] = jnp.concatenate(idxs, axis=0)
    wts_ref[0, 0] = jnp.concatenate(ws, axis=0)


def _sc_gather_body(h1t_hbm, idx_hbm, wts_hbm, out_hbm, h1_v, idx_v, wt_v, out_v):
    num_cores = 2
    cid = lax.axis_index("c")
    sid = lax.axis_index("s")
    wid = sid * num_cores + cid
    b = wid // 4
    part = wid % 4

    pltpu.sync_copy(h1t_hbm.at[b], h1_v)
    pltpu.sync_copy(idx_hbm.at[b, 2 * part], idx_v.at[0])
    pltpu.sync_copy(idx_hbm.at[b, 2 * part + 1], idx_v.at[1])
    pltpu.sync_copy(wts_hbm.at[b, 2 * part], wt_v.at[0])
    pltpu.sync_copy(wts_hbm.at[b, 2 * part + 1], wt_v.at[1])

    for half in range(2):
        def group(i, _, half=half):
            col = i * 16
            i1 = idx_v[half, 0, pl.ds(col, 16)]
            i2 = idx_v[half, 1, pl.ds(col, 16)]
            i3 = idx_v[half, 2, pl.ds(col, 16)]
            w1 = wt_v[half, 0, pl.ds(col, 16)]
            w2 = wt_v[half, 1, pl.ds(col, 16)]
            w3 = wt_v[half, 2, pl.ds(col, 16)]
            for c in range(12):
                base = jnp.int32(c * 1024)
                g1 = plsc.load_gather(h1_v, [base + i1])
                g2 = plsc.load_gather(h1_v, [base + i2])
                g3 = plsc.load_gather(h1_v, [base + i3])
                out_v[c, pl.ds(half * 512 + col, 16)] = w1 * g1 + w2 * g2 + w3 * g3
            return 0

        lax.fori_loop(0, 32, group, 0)
    pltpu.sync_copy(out_v, out_hbm.at[b, part])


def _mlp_body(interp_ref, w2t_ref, b2t_ref, wf1t_ref, bf1t_ref,
              wf2t_ref, bf2t_ref, wf3t_ref, bf3t_ref, x1_ref, x2_ref):
    interp = interp_ref[0, 0]
    h = jnp.maximum(
        jnp.dot(w2t_ref[...], interp, preferred_element_type=jnp.float32)
        + b2t_ref[...], 0.0)
    hf = jnp.maximum(
        jnp.dot(wf1t_ref[...], h, preferred_element_type=jnp.float32)
        + bf1t_ref[...], 0.0)
    x1_ref[0, 0] = jnp.maximum(
        jnp.dot(wf2t_ref[...], hf, preferred_element_type=jnp.float32)
        + bf2t_ref[...], 0.0)
    x2_ref[0, 0] = jnp.maximum(
        jnp.dot(wf3t_ref[...], hf, preferred_element_type=jnp.float32)
        + bf3t_ref[...], 0.0)


def kernel(xyz0, xyz1, xyz2, x, W1, b1, W2, b2, Wf1, bf1, Wf2, bf2, Wf3, bf3):
    B, N0, _ = xyz0.shape
    N1 = xyz1.shape[1]
    N2 = xyz2.shape[1]
    BLK = 512
    NCH = N0 // BLK

    x0t = jnp.swapaxes(xyz0, 1, 2)
    x1t = jnp.swapaxes(xyz1, 1, 2)
    xt = jnp.swapaxes(x, 1, 2)
    w1t, w2t = W1.T, W2.T
    wf1t, wf2t, wf3t = Wf1.T, Wf2.T, Wf3.T
    b1t = b1.reshape(-1, 1)
    b2t = b2.reshape(-1, 1)
    bf1t = bf1.reshape(-1, 1)
    bf2t = bf2.reshape(-1, 1)
    bf3t = bf3.reshape(-1, 1)

    h1t, idx3, wts3 = pl.pallas_call(
        _select_body,
        grid=(B, NCH),
        in_specs=[
            pl.BlockSpec((1, 3, N1), lambda b, j: (b, 0, 0)),
            pl.BlockSpec((1, N2, 3), lambda b, j: (b, 0, 0)),
            pl.BlockSpec((1, 6, N2), lambda b, j: (b, 0, 0)),
            pl.BlockSpec(w1t.shape, lambda b, j: (0, 0)),
            pl.BlockSpec((12, 1), lambda b, j: (0, 0)),
            pl.BlockSpec((1, 3, BLK), lambda b, j: (b, 0, j)),
            pl.BlockSpec((1, N1, 3), lambda b, j: (b, 0, 0)),
        ],
        out_specs=[
            pl.BlockSpec((1, 12, N1), lambda b, j: (b, 0, 0)),
            pl.BlockSpec((1, 1, 3, BLK), lambda b, j: (b, j, 0, 0)),
            pl.BlockSpec((1, 1, 3, BLK), lambda b, j: (b, j, 0, 0)),
        ],
        out_shape=[
            jax.ShapeDtypeStruct((B, 12, N1), jnp.float32),
            jax.ShapeDtypeStruct((B, NCH, 3, BLK), jnp.int32),
            jax.ShapeDtypeStruct((B, NCH, 3, BLK), jnp.float32),
        ],
    )(x1t, xyz2, xt, w1t, b1t, x0t, xyz1)

    h1flat = h1t.reshape(B, 12 * N1)

    mesh = plsc.VectorSubcoreMesh(core_axis_name="c", subcore_axis_name="s")
    sc_gather = pl.kernel(
        _sc_gather_body,
        out_type=jax.ShapeDtypeStruct((B, 4, 12, 1024), jnp.float32),
        mesh=mesh,
        compiler_params=pltpu.CompilerParams(needs_layout_passes=False),
        scratch_types=[
            pltpu.VMEM((12 * N1,), jnp.float32),
            pltpu.VMEM((2, 3, BLK), jnp.int32),
            pltpu.VMEM((2, 3, BLK), jnp.float32),
            pltpu.VMEM((12, 1024), jnp.float32),
        ],
    )
    interp = sc_gather(h1flat, idx3, wts3)

    x1o, x2o = pl.pallas_call(
        _mlp_body,
        grid=(B, 4),
        in_specs=[
            pl.BlockSpec((1, 1, 12, 1024), lambda b, p: (b, p, 0, 0)),
            pl.BlockSpec(w2t.shape, lambda b, p: (0, 0)),
            pl.BlockSpec((12, 1), lambda b, p: (0, 0)),
            pl.BlockSpec(wf1t.shape, lambda b, p: (0, 0)),
            pl.BlockSpec((24, 1), lambda b, p: (0, 0)),
            pl.BlockSpec(wf2t.shape, lambda b, p: (0, 0)),
            pl.BlockSpec((8, 1), lambda b, p: (0, 0)),
            pl.BlockSpec(wf3t.shape, lambda b, p: (0, 0)),
            pl.BlockSpec((8, 1), lambda b, p: (0, 0)),
        ],
        out_specs=[
            pl.BlockSpec((1, 1, 8, 1024), lambda b, p: (b, p, 0, 0)),
            pl.BlockSpec((1, 1, 8, 1024), lambda b, p: (b, p, 0, 0)),
        ],
        out_shape=[
            jax.ShapeDtypeStruct((B, 4, 8, 1024), jnp.float32),
            jax.ShapeDtypeStruct((B, 4, 8, 1024), jnp.float32),
        ],
    )(interp, w2t, b2t, wf1t, bf1t, wf2t, bf2t, wf3t, bf3t)

    x1f = x1o.transpose(0, 1, 3, 2).reshape(B, N0, 8)
    x2f = x2o.transpose(0, 1, 3, 2).reshape(B, N0, 8)
    return (x1f, x2f)


# speedup vs baseline: 1.0010x; 1.0010x over previous
import jax
import jax.numpy as jnp
from jax import lax
from jax.experimental import pallas as pl
from jax.experimental.pallas import tpu as pltpu
from jax.experimental.pallas import tpu_sc as plsc


def _top3_t(d):
    s, n = d.shape
    iota = lax.broadcasted_iota(jnp.int32, (s, n), 0)
    fbig = jnp.float32(3.0e38)
    sbig = jnp.int32(2 * s)
    dwork = d
    idxs, recips, sels = [], [], []
    for k in range(3):
        m = jnp.min(dwork, axis=0, keepdims=True)
        elig = dwork == m
        idxm = jnp.min(jnp.where(elig, iota, sbig), axis=0, keepdims=True)
        sel = iota == idxm
        idxs.append(idxm)
        sels.append(sel)
        recips.append(1.0 / (jnp.maximum(m, 0.0) + 1e-8))
        if k < 2:
            dwork = jnp.where(sel, fbig, dwork)
    rnorm = 1.0 / (recips[0] + recips[1] + recips[2])
    ws = [r * rnorm for r in recips]
    return idxs, ws, sels


def _weight_matrix(sels, ws, like):
    w = jnp.where(sels[0], ws[0], jnp.zeros_like(like))
    w = jnp.where(sels[1], ws[1], w)
    w = jnp.where(sels[2], ws[2], w)
    return w


def _sqdist_t(r, qt):
    rr = jnp.sum(r * r, axis=-1, keepdims=True)
    qq = jnp.sum(qt * qt, axis=0, keepdims=True)
    rq = jnp.dot(r, qt, preferred_element_type=jnp.float32)
    return rr + qq - 2.0 * rq


def _select_body(x1t_ref, xyz2_ref, xt_ref, w1t_ref, b1t_ref,
                 x0t_ref, xyz1_ref, h1t_ref, idx_ref, wts_ref, h1t_s):
    j = pl.program_id(1)

    @pl.when(j == 0)
    def _stage1():
        d = _sqdist_t(xyz2_ref[0], x1t_ref[0])
        _, ws, sels = _top3_t(d)
        w = _weight_matrix(sels, ws, d)
        interp = jnp.dot(xt_ref[0], w, preferred_element_type=jnp.float32)
        h = jnp.dot(w1t_ref[...], interp, preferred_element_type=jnp.float32)
        h1t_s[...] = jnp.maximum(h + b1t_ref[...], 0.0)

    h1t_ref[0] = h1t_s[...]

    d = _sqdist_t(xyz1_ref[0], x0t_ref[0])
    idxs, ws, _ = _top3_t(d)
    idx_ref[0, 0] = jnp.concatenate(idxs, axis=0)
    wts_ref[0, 0] = jnp.concatenate(ws, axis=0)


def _sc_gather_body(h1t_hbm, idx_hbm, wts_hbm, out_hbm, h1_v, idx_v, wt_v, out_v):
    num_cores = 2
    cid = lax.axis_index("c")
    sid = lax.axis_index("s")
    wid = sid * num_cores + cid
    b = wid // 4
    part = wid % 4

    pltpu.sync_copy(h1t_hbm.at[b], h1_v)
    pltpu.sync_copy(idx_hbm.at[b, 2 * part], idx_v.at[0])
    pltpu.sync_copy(idx_hbm.at[b, 2 * part + 1], idx_v.at[1])
    pltpu.sync_copy(wts_hbm.at[b, 2 * part], wt_v.at[0])
    pltpu.sync_copy(wts_hbm.at[b, 2 * part + 1], wt_v.at[1])

    for half in range(2):
        def group(i, _, half=half):
            col = i * 16
            i1 = idx_v[half, 0, pl.ds(col, 16)]
            i2 = idx_v[half, 1, pl.ds(col, 16)]
            i3 = idx_v[half, 2, pl.ds(col, 16)]
            w1 = wt_v[half, 0, pl.ds(col, 16)]
            w2 = wt_v[half, 1, pl.ds(col, 16)]
            w3 = wt_v[half, 2, pl.ds(col, 16)]
            for c in range(12):
                base = jnp.int32(c * 1024)
                g1 = plsc.load_gather(h1_v, [base + i1])
                g2 = plsc.load_gather(h1_v, [base + i2])
                g3 = plsc.load_gather(h1_v, [base + i3])
                out_v[c, pl.ds(half * 512 + col, 16)] = w1 * g1 + w2 * g2 + w3 * g3
            return 0

        lax.fori_loop(0, 32, group, 0)
    pltpu.sync_copy(out_v, out_hbm.at[b, part])


def _mlp_body(interp_ref, w2t_ref, b2t_ref, wf1t_ref, bf1t_ref,
              wf2t_ref, bf2t_ref, wf3t_ref, bf3t_ref, x1_ref, x2_ref):
    interp = interp_ref[0, 0]
    h = jnp.maximum(
        jnp.dot(w2t_ref[...], interp, preferred_element_type=jnp.float32)
        + b2t_ref[...], 0.0)
    hf = jnp.maximum(
        jnp.dot(wf1t_ref[...], h, preferred_element_type=jnp.float32)
        + bf1t_ref[...], 0.0)
    x1_ref[0, 0] = jnp.maximum(
        jnp.dot(wf2t_ref[...], hf, preferred_element_type=jnp.float32)
        + bf2t_ref[...], 0.0)
    x2_ref[0, 0] = jnp.maximum(
        jnp.dot(wf3t_ref[...], hf, preferred_element_type=jnp.float32)
        + bf3t_ref[...], 0.0)


def kernel(xyz0, xyz1, xyz2, x, W1, b1, W2, b2, Wf1, bf1, Wf2, bf2, Wf3, bf3):
    B, N0, _ = xyz0.shape
    N1 = xyz1.shape[1]
    N2 = xyz2.shape[1]
    BLK = 512
    NCH = N0 // BLK

    x0t = jnp.swapaxes(xyz0, 1, 2)
    x1t = jnp.swapaxes(xyz1, 1, 2)
    xt = jnp.swapaxes(x, 1, 2)
    w1t, w2t = W1.T, W2.T
    wf1t, wf2t, wf3t = Wf1.T, Wf2.T, Wf3.T
    b1t = b1.reshape(-1, 1)
    b2t = b2.reshape(-1, 1)
    bf1t = bf1.reshape(-1, 1)
    bf2t = bf2.reshape(-1, 1)
    bf3t = bf3.reshape(-1, 1)

    h1t, idx3, wts3 = pl.pallas_call(
        _select_body,
        grid=(B, NCH),
        in_specs=[
            pl.BlockSpec((1, 3, N1), lambda b, j: (b, 0, 0)),
            pl.BlockSpec((1, N2, 3), lambda b, j: (b, 0, 0)),
            pl.BlockSpec((1, 6, N2), lambda b, j: (b, 0, 0)),
            pl.BlockSpec(w1t.shape, lambda b, j: (0, 0)),
            pl.BlockSpec((12, 1), lambda b, j: (0, 0)),
            pl.BlockSpec((1, 3, BLK), lambda b, j: (b, 0, j)),
            pl.BlockSpec((1, N1, 3), lambda b, j: (b, 0, 0)),
        ],
        out_specs=[
            pl.BlockSpec((1, 12, N1), lambda b, j: (b, 0, 0)),
            pl.BlockSpec((1, 1, 3, BLK), lambda b, j: (b, j, 0, 0)),
            pl.BlockSpec((1, 1, 3, BLK), lambda b, j: (b, j, 0, 0)),
        ],
        out_shape=[
            jax.ShapeDtypeStruct((B, 12, N1), jnp.float32),
            jax.ShapeDtypeStruct((B, NCH, 3, BLK), jnp.int32),
            jax.ShapeDtypeStruct((B, NCH, 3, BLK), jnp.float32),
        ],
        scratch_shapes=[pltpu.VMEM((12, N1), jnp.float32)],
    )(x1t, xyz2, xt, w1t, b1t, x0t, xyz1)

    h1flat = h1t.reshape(B, 12 * N1)

    mesh = plsc.VectorSubcoreMesh(core_axis_name="c", subcore_axis_name="s")
    sc_gather = pl.kernel(
        _sc_gather_body,
        out_type=jax.ShapeDtypeStruct((B, 4, 12, 1024), jnp.float32),
        mesh=mesh,
        compiler_params=pltpu.CompilerParams(needs_layout_passes=False),
        scratch_types=[
            pltpu.VMEM((12 * N1,), jnp.float32),
            pltpu.VMEM((2, 3, BLK), jnp.int32),
            pltpu.VMEM((2, 3, BLK), jnp.float32),
            pltpu.VMEM((12, 1024), jnp.float32),
        ],
    )
    interp = sc_gather(h1flat, idx3, wts3)

    x1o, x2o = pl.pallas_call(
        _mlp_body,
        grid=(B, 4),
        in_specs=[
            pl.BlockSpec((1, 1, 12, 1024), lambda b, p: (b, p, 0, 0)),
            pl.BlockSpec(w2t.shape, lambda b, p: (0, 0)),
            pl.BlockSpec((12, 1), lambda b, p: (0, 0)),
            pl.BlockSpec(wf1t.shape, lambda b, p: (0, 0)),
            pl.BlockSpec((24, 1), lambda b, p: (0, 0)),
            pl.BlockSpec(wf2t.shape, lambda b, p: (0, 0)),
            pl.BlockSpec((8, 1), lambda b, p: (0, 0)),
            pl.BlockSpec(wf3t.shape, lambda b, p: (0, 0)),
            pl.BlockSpec((8, 1), lambda b, p: (0, 0)),
        ],
        out_specs=[
            pl.BlockSpec((1, 1, 8, 1024), lambda b, p: (b, p, 0, 0)),
            pl.BlockSpec((1, 1, 8, 1024), lambda b, p: (b, p, 0, 0)),
        ],
        out_shape=[
            jax.ShapeDtypeStruct((B, 4, 8, 1024), jnp.float32),
            jax.ShapeDtypeStruct((B, 4, 8, 1024), jnp.float32),
        ],
    )(interp, w2t, b2t, wf1t, bf1t, wf2t, bf2t, wf3t, bf3t)

    x1f = x1o.transpose(0, 1, 3, 2).reshape(B, N0, 8)
    x2f = x2o.transpose(0, 1, 3, 2).reshape(B, N0, 8)
    return (x1f, x2f)
